# baseline (speedup 1.0000x reference)
import functools

import jax
import jax.numpy as jnp
from jax import lax
from jax.experimental import pallas as pl
from jax.experimental.pallas import tpu as pltpu
from jax.experimental.pallas import tpu_sc as plsc

_DEPTH = 1000
_N = 4096
_J = 20
_JPAD = 32
_ROW = _DEPTH * _J
_NC = 2
_NS = 16
_NW = _NC * _NS
_RPW = _N // _NW
_R = 2
_NBUF = 2
_G = _RPW // _R
_BLK = _R * _ROW


def _body(x_hbm, out_hbm, xv, buf0, buf1, sem0, sem1):
    bufs = (buf0, buf1)
    sems = (sem0, sem1)
    wid = lax.axis_index("s") * _NC + lax.axis_index("c")
    base = wid * _RPW

    iota = lax.iota(jnp.int32, 16)
    ones_v = jnp.full((16,), 1.0, jnp.float32)
    zeros_v = jnp.zeros((16,), jnp.float32)
    hi_mask = iota < (_J - 16)

    pltpu.sync_copy(x_hbm.at[pl.ds(base * _JPAD, _RPW * _JPAD)], xv)

    def _zero(i, carry):
        for b in range(_NBUF):
            bufs[b][pl.ds(i * 16, 16)] = zeros_v
        return carry

    lax.fori_loop(0, _BLK // 16, _zero, 0)

    def _scatter(b, g, val):
        for r in range(_R):
            lrow = g * _R + r
            x_lo = xv[pl.ds(lrow * _JPAD, 16)]
            x_hi = xv[pl.ds(lrow * _JPAD + 16, 16)]
            off_lo = x_lo * _J + iota + (r * _ROW)
            off_hi = x_hi * _J + (iota + (16 + r * _ROW))
            plsc.store_scatter(bufs[b], [off_lo], val)
            plsc.store_scatter(bufs[b], [off_hi], val, mask=hi_mask)

    def _start(b, g):
        dst = out_hbm.at[pl.ds((base + g * _R) * _ROW, _BLK)]
        pltpu.async_copy(bufs[b], dst, sems[b])

    def _wait(b):
        pltpu.make_async_copy(bufs[b], out_hbm.at[pl.ds(0, _BLK)], sems[b]).wait()

    for b in range(_NBUF):
        _scatter(b, b, ones_v)
        _start(b, b)

    def _step(it, carry):
        for b in range(_NBUF):
            g = it * _NBUF + b
            _wait(b)
            _scatter(b, g - _NBUF, zeros_v)
            _scatter(b, g, ones_v)
            _start(b, g)
        return carry

    lax.fori_loop(1, _G // _NBUF, _step, 0)

    for b in range(_NBUF):
        _wait(b)


@jax.jit
def _one_hot_sc(x_pad_flat):
    mesh = plsc.VectorSubcoreMesh(core_axis_name="c", subcore_axis_name="s")
    f = pl.kernel(
        _body,
        out_type=jax.ShapeDtypeStruct((_N * _ROW,), jnp.float32),
        mesh=mesh,
        compiler_params=pltpu.CompilerParams(
            needs_layout_passes=False, use_tc_tiling_on_sc=False
        ),
        scratch_types=[
            pltpu.VMEM((_RPW * _JPAD,), jnp.int32),
            pltpu.VMEM((_BLK,), jnp.float32),
            pltpu.VMEM((_BLK,), jnp.float32),
            pltpu.SemaphoreType.DMA,
            pltpu.SemaphoreType.DMA,
        ],
    )
    return f(x_pad_flat)


def kernel(X_in, ones):
    del ones
    x = X_in.astype(jnp.int32)
    x_pad = jnp.pad(x, ((0, 0), (0, _JPAD - _J)))
    out = _one_hot_sc(x_pad.reshape(-1))
    return out.reshape(_N, _DEPTH, _J)


# speedup vs baseline: 43.3137x; 43.3137x over previous
import jax
import jax.numpy as jnp
from jax import lax
from jax.experimental import pallas as pl
from jax.experimental.pallas import tpu as pltpu
from jax.experimental.pallas import tpu_sc as plsc

_DEPTH = 1000
_N = 4096
_J = 20
_NC = 2
_NS = 16
_NW = _NC * _NS
_IW = _N // _NW
_DC = 200
_NCHUNK = _DEPTH // _DC


def _body(xt_hbm, out_hbm, xv, b0, b1, b2, b3, b4, s0, s1, s2, s3, s4):
    bufs = (b0, b1, b2, b3, b4)
    sems = (s0, s1, s2, s3, s4)
    wid = lax.axis_index("s") * _NC + lax.axis_index("c")

    iota = lax.iota(jnp.int32, 16)
    ones_v = jnp.full((16,), 1.0, jnp.float32)
    zeros_v = jnp.zeros((16,), jnp.float32)

    pltpu.sync_copy(xt_hbm.at[pl.ds(wid * (_J * _IW), _J * _IW)], xv)

    def _zero(i, carry):
        for b in range(_NCHUNK):
            for v in range(_IW // 16):
                bufs[b][i, pl.ds(v * 16, 16)] = zeros_v
        return carry

    lax.fori_loop(0, _DC, _zero, 0)

    def _scatter(c, j, val):
        d0 = c * _DC
        for v in range(_IW // 16):
            xi = xv[pl.ds(j * _IW + v * 16, 16)]
            m = (xi >= d0) & (xi < d0 + _DC)
            row = jnp.where(m, xi - d0, 0)
            col = iota + (v * 16)
            plsc.store_scatter(bufs[c], [row, col], val, mask=m)

    def _start(c, j):
        dst = out_hbm.at[j, pl.ds(c * _DC, _DC), pl.ds(wid * _IW, _IW)]
        pltpu.async_copy(bufs[c], dst, sems[c])

    def _wait(c):
        dst = out_hbm.at[0, pl.ds(0, _DC), pl.ds(0, _IW)]
        pltpu.make_async_copy(bufs[c], dst, sems[c]).wait()

    for c in range(_NCHUNK):
        _scatter(c, 0, ones_v)
        _start(c, 0)

    def _step(j, carry):
        for c in range(_NCHUNK):
            _wait(c)
            _scatter(c, j - 1, zeros_v)
            _scatter(c, j, ones_v)
            _start(c, j)
        return carry

    lax.fori_loop(1, _J, _step, 0)

    for c in range(_NCHUNK):
        _wait(c)


@jax.jit
def _one_hot_sc(xtr):
    mesh = plsc.VectorSubcoreMesh(core_axis_name="c", subcore_axis_name="s")
    f = pl.kernel(
        _body,
        out_type=jax.ShapeDtypeStruct((_J, _DEPTH, _N), jnp.float32),
        mesh=mesh,
        compiler_params=pltpu.CompilerParams(
            needs_layout_passes=False, use_tc_tiling_on_sc=True
        ),
        scratch_types=[pltpu.VMEM((_J * _IW,), jnp.int32)]
        + [pltpu.VMEM((_DC, _IW), jnp.float32) for _ in range(_NCHUNK)]
        + [pltpu.SemaphoreType.DMA for _ in range(_NCHUNK)],
    )
    return f(xtr)


def kernel(X_in, ones):
    del ones
    xt = X_in.astype(jnp.int32).T.reshape(_J, _NW, _IW)
    xtr = jnp.transpose(xt, (1, 0, 2)).reshape(-1)
    out = _one_hot_sc(xtr)
    return jnp.transpose(out, (2, 1, 0))


# speedup vs baseline: 44.6803x; 1.0316x over previous
import jax
import jax.numpy as jnp
from jax import lax
from jax.experimental import pallas as pl
from jax.experimental.pallas import tpu as pltpu
from jax.experimental.pallas import tpu_sc as plsc

_DEPTH = 1000
_N = 4096
_J = 20
_NC = 2
_NS = 16
_NW = _NC * _NS
_IW = _N // _NW
_DC = 200
_NCHUNK = _DEPTH // _DC


def _body(xt_hbm, out_hbm, xv, b0, b1, b2, b3, b4, s0, s1, s2, s3, s4):
    bufs = (b0, b1, b2, b3, b4)
    sems = (s0, s1, s2, s3, s4)
    wid = lax.axis_index("s") * _NC + lax.axis_index("c")

    iota = lax.iota(jnp.int32, 16)
    ones_v = jnp.full((16,), 1.0, jnp.float32)
    zeros_v = jnp.zeros((16,), jnp.float32)

    pltpu.sync_copy(xt_hbm.at[pl.ds(wid * (_J * _IW), _J * _IW)], xv)

    def _scatter(c, j, val):
        d0 = c * _DC
        for v in range(_IW // 16):
            xi = xv[pl.ds(j * _IW + v * 16, 16)]
            u = xi - d0
            m = u.astype(jnp.uint32) < jnp.uint32(_DC)
            row = jnp.where(m, u, 0)
            col = iota + (v * 16)
            plsc.store_scatter(bufs[c], [row, col], val, mask=m)

    def _start(c, j):
        dst = out_hbm.at[j, pl.ds(c * _DC, _DC), pl.ds(wid * _IW, _IW)]
        pltpu.async_copy(bufs[c], dst, sems[c])

    def _wait(c):
        dst = out_hbm.at[0, pl.ds(0, _DC), pl.ds(0, _IW)]
        pltpu.make_async_copy(bufs[c], dst, sems[c]).wait()

    def _zero(c):
        def zb(i, carry):
            for v in range(_IW // 16):
                bufs[c][i, pl.ds(v * 16, 16)] = zeros_v
            return carry

        lax.fori_loop(0, _DC, zb, 0)

    for c in range(_NCHUNK):
        _zero(c)
        _scatter(c, 0, ones_v)
        _start(c, 0)

    def _step(j, carry):
        for c in range(_NCHUNK):
            _wait(c)
            _scatter(c, j - 1, zeros_v)
            _scatter(c, j, ones_v)
            _start(c, j)
        return carry

    lax.fori_loop(1, _J, _step, 0)

    for c in range(_NCHUNK):
        _wait(c)


@jax.jit
def _one_hot_sc(xtr):
    mesh = plsc.VectorSubcoreMesh(core_axis_name="c", subcore_axis_name="s")
    f = pl.kernel(
        _body,
        out_type=jax.ShapeDtypeStruct((_J, _DEPTH, _N), jnp.float32),
        mesh=mesh,
        compiler_params=pltpu.CompilerParams(
            needs_layout_passes=False, use_tc_tiling_on_sc=True
        ),
        scratch_types=[pltpu.VMEM((_J * _IW,), jnp.int32)]
        + [pltpu.VMEM((_DC, _IW), jnp.float32) for _ in range(_NCHUNK)]
        + [pltpu.SemaphoreType.DMA for _ in range(_NCHUNK)],
    )
    return f(xtr)


def kernel(X_in, ones):
    del ones
    x3 = X_in.astype(jnp.int32).reshape(_NW, _IW, _J)
    xtr = jnp.transpose(x3, (0, 2, 1)).reshape(-1)
    out = _one_hot_sc(xtr)
    return jnp.transpose(out, (2, 1, 0))


# speedup vs baseline: 45.4481x; 1.0172x over previous
import jax
import jax.numpy as jnp
from jax import lax
from jax.experimental import pallas as pl
from jax.experimental.pallas import tpu as pltpu
from jax.experimental.pallas import tpu_sc as plsc

_DEPTH = 1000
_N = 4096
_J = 20
_NC = 2
_NS = 16
_NW = _NC * _NS
_IW = _N // _NW
_DC = 200
_NCHUNK = _DEPTH // _DC
_NUNIT = _J * _NCHUNK
_NBUF = 4


def _div5(g):
    return lax.shift_right_logical(g * 52429, 18)


def _body(x_hbm, out_hbm, xs, b0, b1, b2, b3, s0, s1, s2, s3):
    bufs = (b0, b1, b2, b3)
    sems = (s0, s1, s2, s3)
    wid = lax.axis_index("s") * _NC + lax.axis_index("c")

    iota = lax.iota(jnp.int32, 16)
    ones_v = jnp.full((16,), 1.0, jnp.float32)
    zeros_v = jnp.zeros((16,), jnp.float32)

    pltpu.sync_copy(x_hbm.at[pl.ds(wid * _IW, _IW), :], xs)

    def _scatter(b, g, val):
        j = _div5(g)
        d0 = (g - j * _NCHUNK) * _DC
        jv = jnp.zeros((16,), jnp.int32) + j
        for v in range(_IW // 16):
            col = iota + (v * 16)
            xi = plsc.load_gather(xs, [col, jv])
            u = xi - d0
            m = u.astype(jnp.uint32) < jnp.uint32(_DC)
            row = jnp.where(m, u, 0)
            plsc.store_scatter(bufs[b], [row, col], val, mask=m)

    def _start(b, g):
        j = _div5(g)
        d0 = (g - j * _NCHUNK) * _DC
        dst = out_hbm.at[j, pl.ds(d0, _DC), pl.ds(wid * _IW, _IW)]
        pltpu.async_copy(bufs[b], dst, sems[b])

    def _wait(b):
        dst = out_hbm.at[0, pl.ds(0, _DC), pl.ds(0, _IW)]
        pltpu.make_async_copy(bufs[b], dst, sems[b]).wait()

    def _zero(b):
        def zb(i, carry):
            for v in range(_IW // 16):
                bufs[b][i, pl.ds(v * 16, 16)] = zeros_v
            return carry

        lax.fori_loop(0, _DC, zb, 0)

    for b in range(_NBUF):
        _zero(b)
        _scatter(b, b, ones_v)
        _start(b, b)

    def _step(it, carry):
        for b in range(_NBUF):
            g = it * _NBUF + b
            _wait(b)
            _scatter(b, g - _NBUF, zeros_v)
            _scatter(b, g, ones_v)
            _start(b, g)
        return carry

    lax.fori_loop(1, _NUNIT // _NBUF, _step, 0)

    for b in range(_NBUF):
        _wait(b)


@jax.jit
def _one_hot_sc(x2d):
    mesh = plsc.VectorSubcoreMesh(core_axis_name="c", subcore_axis_name="s")
    f = pl.kernel(
        _body,
        out_type=jax.ShapeDtypeStruct((_J, _DEPTH, _N), jnp.float32),
        mesh=mesh,
        compiler_params=pltpu.CompilerParams(
            needs_layout_passes=False, use_tc_tiling_on_sc=True
        ),
        scratch_types=[pltpu.VMEM((_IW, _J), jnp.int32)]
        + [pltpu.VMEM((_DC, _IW), jnp.float32) for _ in range(_NBUF)]
        + [pltpu.SemaphoreType.DMA for _ in range(_NBUF)],
    )
    return f(x2d)


def kernel(X_in, ones):
    del ones
    out = _one_hot_sc(X_in.astype(jnp.int32))
    return jnp.transpose(out, (2, 1, 0))
